# baseline (speedup 1.0000x reference)
import functools

import jax
import jax.numpy as jnp
from jax import lax
from jax.experimental import pallas as pl
from jax.experimental.pallas import tpu as pltpu
from jax.experimental.pallas import tpu_sc as plsc

F_DIM = 64
HIST = 50
MU = 3.5
NORM = float(HIST) ** (-0.5)
NCHUNK = F_DIM // 16


def _svdpp_body(user_hbm, item_hbm, imp_hbm, ub_hbm, ib_hbm, P_hbm, Q_hbm,
                Y_hbm, out_hbm, user_v, item_v, imp_v, bu_v, bi_v, pu_v,
                qi_v, rows_v, res_v, sem0, sem1, sem2, sem3, sem4):
    cid = lax.axis_index("c")
    sid = lax.axis_index("s")

    @pl.when(jnp.logical_and(cid == 0, sid == 0))
    def _():
        c0 = pltpu.async_copy(user_hbm, user_v, sem0)
        c1 = pltpu.async_copy(item_hbm, item_v, sem1)
        c2 = pltpu.async_copy(imp_hbm, imp_v, sem2)
        c0.wait()
        c1.wait()
        c2.wait()
        g0 = pltpu.async_copy(ub_hbm.at[user_v], bu_v.at[pl.ds(0, 1)], sem0)
        g1 = pltpu.async_copy(ib_hbm.at[item_v], bi_v.at[pl.ds(0, 1)], sem1)
        g2 = pltpu.async_copy(P_hbm.at[user_v], pu_v, sem2)
        g3 = pltpu.async_copy(Q_hbm.at[item_v], qi_v, sem3)
        g4 = pltpu.async_copy(Y_hbm.at[imp_v], rows_v, sem4)
        g0.wait()
        g1.wait()
        g2.wait()
        g3.wait()
        g4.wait()

        total = None
        for c in range(NCHUNK):
            acc = rows_v[0, c * 16:(c + 1) * 16]
            for j in range(1, HIST):
                acc = acc + rows_v[j, c * 16:(c + 1) * 16]
            p = pu_v[0, c * 16:(c + 1) * 16]
            q = qi_v[0, c * 16:(c + 1) * 16]
            t = p * (q + NORM * acc)
            total = t if total is None else total + t
        s = total[0]
        for i in range(1, 16):
            s = s + total[i]
        bu = bu_v[...][0]
        bi = bi_v[...][0]
        r = MU + bu + bi + s
        res_v[...] = jnp.full((16,), r, jnp.float32)
        pltpu.sync_copy(res_v.at[pl.ds(0, 1)], out_hbm)


_svdpp = functools.partial(
    pl.kernel,
    out_type=jax.ShapeDtypeStruct((1,), jnp.float32),
    mesh=plsc.VectorSubcoreMesh(core_axis_name="c", subcore_axis_name="s"),
    compiler_params=pltpu.CompilerParams(use_tc_tiling_on_sc=False),
    scratch_types=[
        pltpu.VMEM((1,), jnp.int32),
        pltpu.VMEM((1,), jnp.int32),
        pltpu.VMEM((HIST,), jnp.int32),
        pltpu.VMEM((16,), jnp.float32),
        pltpu.VMEM((16,), jnp.float32),
        pltpu.VMEM((1, F_DIM), jnp.float32),
        pltpu.VMEM((1, F_DIM), jnp.float32),
        pltpu.VMEM((HIST, F_DIM), jnp.float32),
        pltpu.VMEM((16,), jnp.float32),
        pltpu.SemaphoreType.DMA,
        pltpu.SemaphoreType.DMA,
        pltpu.SemaphoreType.DMA,
        pltpu.SemaphoreType.DMA,
        pltpu.SemaphoreType.DMA,
    ],
)(_svdpp_body)


def kernel(user, item, implicit_items, user_biases, item_biases, P, Q, Y):
    return _svdpp(
        user.astype(jnp.int32),
        item.astype(jnp.int32),
        implicit_items.astype(jnp.int32),
        user_biases.reshape(-1),
        item_biases.reshape(-1),
        P,
        Q,
        Y,
    )


# speedup vs baseline: 1.9923x; 1.9923x over previous
import functools

import jax
import jax.numpy as jnp
from jax import lax
from jax.experimental import pallas as pl
from jax.experimental.pallas import tpu as pltpu
from jax.experimental.pallas import tpu_sc as plsc

F_DIM = 64
HIST = 50
MU = 3.5
NORM = float(HIST) ** (-0.5)
NCHUNK = F_DIM // 16
NGRP = (HIST + 15) // 16 * 16


def _svdpp_body(user_hbm, item_hbm, imp_hbm, ub_hbm, ib_hbm, P_hbm, Q_hbm,
                Y_hbm, out_hbm, user_v, item_v, imp_v, bu_v, bi_v,
                pu_v, qi_v, rows_v, res_v, sem0, sem1, sem2, sem3):
    cid = lax.axis_index("c")
    sid = lax.axis_index("s")

    @pl.when(jnp.logical_and(cid == 0, sid == 0))
    def _():
        c0 = pltpu.async_copy(user_hbm, user_v.at[pl.ds(0, 1)], sem0)
        c1 = pltpu.async_copy(item_hbm, item_v.at[pl.ds(0, 1)], sem1)
        c2 = pltpu.async_copy(imp_hbm, imp_v.at[pl.ds(0, HIST)], sem2)
        c2.wait()
        impvecs = [imp_v[pl.ds(g * 16, 16)] for g in range(NGRP // 16)]
        c0.wait()
        c1.wait()
        u = user_v[...][0]
        it = item_v[...][0]
        ycopies = []
        for j in range(HIST):
            gj = impvecs[j // 16][j % 16] >> 3
            ycopies.append(pltpu.async_copy(
                Y_hbm.at[pl.ds(gj, 1)], rows_v.at[pl.ds(j, 1)], sem3))
        gP = pltpu.async_copy(P_hbm.at[pl.ds(u >> 3, 1)], pu_v, sem0)
        gQ = pltpu.async_copy(Q_hbm.at[pl.ds(it >> 3, 1)], qi_v, sem1)
        ub_base = pl.multiple_of((u >> 3) << 3, 8)
        ib_base = pl.multiple_of((it >> 3) << 3, 8)
        gb0 = pltpu.async_copy(ub_hbm.at[pl.ds(ub_base, 8)],
                               bu_v.at[pl.ds(0, 8)], sem2)
        gb1 = pltpu.async_copy(ib_hbm.at[pl.ds(ib_base, 8)],
                               bi_v.at[pl.ds(0, 8)], sem2)
        gP.wait()
        gQ.wait()
        gb0.wait()
        gb1.wait()
        for cp in ycopies:
            cp.wait()

        su = u & 7
        si = it & 7
        total = None
        for c in range(NCHUNK):
            sl = slice(c * 16, (c + 1) * 16)
            acc = None
            for j in range(HIST):
                sj = impvecs[j // 16][j % 16] & 7
                row = rows_v[j, sj, sl]
                acc = row if acc is None else acc + row
            p = pu_v[0, su, sl]
            q = qi_v[0, si, sl]
            t = p * (q + NORM * acc)
            total = t if total is None else total + t
        s = total[0]
        for i in range(1, 16):
            s = s + total[i]
        bu = plsc.load_gather(bu_v, [jnp.full((16,), su, jnp.int32)])[0]
        bi = plsc.load_gather(bi_v, [jnp.full((16,), si, jnp.int32)])[0]
        r = MU + bu + bi + s
        res_v[...] = jnp.full((16,), r, jnp.float32)
        pltpu.sync_copy(res_v.at[pl.ds(0, 1)], out_hbm)


_svdpp = functools.partial(
    pl.kernel,
    out_type=jax.ShapeDtypeStruct((1,), jnp.float32),
    mesh=plsc.VectorSubcoreMesh(core_axis_name="c", subcore_axis_name="s"),
    compiler_params=pltpu.CompilerParams(needs_layout_passes=False),
    scratch_types=[
        pltpu.VMEM((16,), jnp.int32),
        pltpu.VMEM((16,), jnp.int32),
        pltpu.VMEM((NGRP,), jnp.int32),
        pltpu.VMEM((16,), jnp.float32),
        pltpu.VMEM((16,), jnp.float32),
        pltpu.VMEM((1, 8, F_DIM), jnp.float32),
        pltpu.VMEM((1, 8, F_DIM), jnp.float32),
        pltpu.VMEM((HIST, 8, F_DIM), jnp.float32),
        pltpu.VMEM((16,), jnp.float32),
        pltpu.SemaphoreType.DMA,
        pltpu.SemaphoreType.DMA,
        pltpu.SemaphoreType.DMA,
        pltpu.SemaphoreType.DMA,
    ],
)(_svdpp_body)


def kernel(user, item, implicit_items, user_biases, item_biases, P, Q, Y):
    n = P.shape[0]
    return _svdpp(
        user.astype(jnp.int32),
        item.astype(jnp.int32),
        implicit_items.astype(jnp.int32),
        user_biases.reshape(-1),
        item_biases.reshape(-1),
        P.reshape(n // 8, 8, F_DIM),
        Q.reshape(n // 8, 8, F_DIM),
        Y.reshape(n // 8, 8, F_DIM),
    )
